# baseline (speedup 1.0000x reference)
import jax
import jax.numpy as jnp
import numpy as np
from jax import lax
from jax.experimental import pallas as pl
from jax.experimental.pallas import tpu as pltpu
from jax.experimental.pallas import tpu_sc as plsc

BATCH = 16384
EMBED = 32
PACK = 128 // EMBED
NUM_CORES = 2
NUM_SUBCORES = 16
NUM_WORKERS = NUM_CORES * NUM_SUBCORES
B_PER_W = BATCH // NUM_WORKERS
CHUNK = 128
NUM_CHUNKS = B_PER_W // CHUNK


def _make_gather_kernel(n_user_rows, n_movie_rows):
    mesh = plsc.VectorSubcoreMesh(core_axis_name="c", subcore_axis_name="s")
    row_t = jax.ShapeDtypeStruct((BATCH, 128), jnp.float32)

    def sc_gather(uq, mq, ut4, mt4):
        @pl.kernel(
            out_type=(row_t, row_t),
            mesh=mesh,
            scratch_types=[
                pltpu.VMEM((B_PER_W,), jnp.int32),
                pltpu.VMEM((B_PER_W,), jnp.int32),
                pltpu.VMEM((B_PER_W, 128), jnp.float32),
                pltpu.SemaphoreType.DMA,
            ],
        )
        def k(u_idx_hbm, m_idx_hbm, ut_hbm, mt_hbm, uo_hbm, mo_hbm,
              ui_v, mi_v, rows_v, sem):
            wid = lax.axis_index("s") * NUM_CORES + lax.axis_index("c")
            base = wid * B_PER_W
            pltpu.sync_copy(u_idx_hbm.at[pl.ds(base, B_PER_W)], ui_v)
            pltpu.sync_copy(m_idx_hbm.at[pl.ds(base, B_PER_W)], mi_v)
            copies = []
            for c in range(NUM_CHUNKS):
                sl = pl.ds(c * CHUNK, CHUNK)
                copies.append(pltpu.async_copy(
                    ut_hbm.at[ui_v.at[sl]], rows_v.at[sl], sem))
            for cp in copies:
                cp.wait()
            pltpu.sync_copy(rows_v, uo_hbm.at[pl.ds(base, B_PER_W)])
            copies = []
            for c in range(NUM_CHUNKS):
                sl = pl.ds(c * CHUNK, CHUNK)
                copies.append(pltpu.async_copy(
                    mt_hbm.at[mi_v.at[sl]], rows_v.at[sl], sem))
            for cp in copies:
                cp.wait()
            pltpu.sync_copy(rows_v, mo_hbm.at[pl.ds(base, B_PER_W)])

        return k(uq, mq, ut4, mt4)

    return sc_gather


_sc_gather = _make_gather_kernel(1, 1)

_SEL = np.zeros((128, PACK), dtype=np.float32)
for _p in range(PACK):
    _SEL[32 * _p:32 * (_p + 1), _p] = 1.0


def _dot_body(ug_ref, mg_ref, wtu_ref, wtm_ref, sel_ref, ohu_ref, ohm_ref,
              b_ref, o_ref):
    su = jnp.dot(ug_ref[...] * wtu_ref[...], sel_ref[...],
                 preferred_element_type=jnp.float32)
    sm = jnp.dot(mg_ref[...] * wtm_ref[...], sel_ref[...],
                 preferred_element_type=jnp.float32)
    o_ref[...] = (jnp.sum(su * ohu_ref[...], axis=1, keepdims=True)
                  + jnp.sum(sm * ohm_ref[...], axis=1, keepdims=True)
                  + b_ref[0, 0])


def kernel(users, movies, user_table, movie_table, fc1_w, fc1_b):
    users = users.astype(jnp.int32)
    movies = movies.astype(jnp.int32)
    ut4 = user_table.reshape(-1, 128)
    mt4 = movie_table.reshape(-1, 128)
    uq = users // PACK
    mq = movies // PACK
    ohu = (users[:, None] % PACK == jnp.arange(PACK)[None, :]).astype(
        jnp.float32)
    ohm = (movies[:, None] % PACK == jnp.arange(PACK)[None, :]).astype(
        jnp.float32)
    ug4, mg4 = _sc_gather(uq, mq, ut4, mt4)
    wtu = jnp.tile(fc1_w[:, :EMBED], (1, PACK))
    wtm = jnp.tile(fc1_w[:, EMBED:], (1, PACK))
    sel = jnp.asarray(_SEL)
    b = fc1_b.reshape(1, 1)
    out = pl.pallas_call(
        _dot_body,
        out_shape=jax.ShapeDtypeStruct((BATCH, 1), jnp.float32),
    )(ug4, mg4, wtu, wtm, sel, ohu, ohm, b)
    return out


# speedup vs baseline: 7.7981x; 7.7981x over previous
import jax
import jax.numpy as jnp
from jax import lax
from jax.experimental import pallas as pl
from jax.experimental.pallas import tpu as pltpu
from jax.experimental.pallas import tpu_sc as plsc

BATCH = 16384
EMBED = 32
NUM_CORES = 2
NUM_SUBCORES = 16
NUM_WORKERS = NUM_CORES * NUM_SUBCORES
B_PER_W = BATCH // NUM_WORKERS
CHUNK = 128
NUM_CHUNKS = B_PER_W // CHUNK
LANES = 16
PROJ_BLOCK = 65536


def _proj_body(t_ref, w_ref, b_ref, o_ref):
    o_ref[...] = jnp.sum(t_ref[...] * w_ref[...], axis=0) + b_ref[0]


def _project(table_t, w_col, bias):
    rows = table_t.shape[1]
    grid = (rows + PROJ_BLOCK - 1) // PROJ_BLOCK
    return pl.pallas_call(
        _proj_body,
        grid=(grid,),
        in_specs=[
            pl.BlockSpec((EMBED, PROJ_BLOCK), lambda i: (0, i)),
            pl.BlockSpec((EMBED, 1), lambda i: (0, 0)),
            pl.BlockSpec(memory_space=pltpu.SMEM),
        ],
        out_specs=pl.BlockSpec((PROJ_BLOCK,), lambda i: (i,)),
        out_shape=jax.ShapeDtypeStruct((rows,), jnp.float32),
    )(table_t, w_col, bias)


def _make_sum_gather():
    mesh = plsc.VectorSubcoreMesh(core_axis_name="c", subcore_axis_name="s")

    def sum_gather(users, movies, proj_u, proj_m):
        @pl.kernel(
            out_type=jax.ShapeDtypeStruct((BATCH,), jnp.float32),
            mesh=mesh,
            scratch_types=[
                pltpu.VMEM((B_PER_W,), jnp.int32),
                pltpu.VMEM((B_PER_W,), jnp.int32),
                pltpu.VMEM((B_PER_W,), jnp.float32),
                pltpu.VMEM((B_PER_W,), jnp.float32),
                pltpu.VMEM((B_PER_W,), jnp.float32),
                pltpu.SemaphoreType.DMA,
            ],
        )
        def k(u_idx_hbm, m_idx_hbm, pu_hbm, pm_hbm, o_hbm,
              ui_v, mi_v, uv_v, mv_v, ov_v, sem):
            wid = lax.axis_index("s") * NUM_CORES + lax.axis_index("c")
            base = wid * B_PER_W
            pltpu.sync_copy(u_idx_hbm.at[pl.ds(base, B_PER_W)], ui_v)
            pltpu.sync_copy(m_idx_hbm.at[pl.ds(base, B_PER_W)], mi_v)
            copies = []
            for c in range(NUM_CHUNKS):
                sl = pl.ds(c * CHUNK, CHUNK)
                copies.append(pltpu.async_copy(
                    pu_hbm.at[ui_v.at[sl]], uv_v.at[sl], sem))
                copies.append(pltpu.async_copy(
                    pm_hbm.at[mi_v.at[sl]], mv_v.at[sl], sem))
            for cp in copies:
                cp.wait()

            @pl.loop(0, B_PER_W, step=LANES)
            def _(i):
                sl = pl.ds(i, LANES)
                ov_v[sl] = uv_v[sl] + mv_v[sl]

            pltpu.sync_copy(ov_v, o_hbm.at[pl.ds(base, B_PER_W)])

        return k(users, movies, proj_u, proj_m)

    return sum_gather


_sum_gather = _make_sum_gather()


def kernel(users, movies, user_table, movie_table, fc1_w, fc1_b):
    users = users.astype(jnp.int32)
    movies = movies.astype(jnp.int32)
    wu = fc1_w[0, :EMBED].reshape(EMBED, 1)
    wm = fc1_w[0, EMBED:].reshape(EMBED, 1)
    zero = jnp.zeros((1,), jnp.float32)
    proj_u = _project(user_table.T, wu, zero)
    proj_m = _project(movie_table.T, wm, fc1_b)
    out = _sum_gather(users, movies, proj_u, proj_m)
    return out.reshape(BATCH, 1)


# speedup vs baseline: 7.8094x; 1.0015x over previous
import jax
import jax.numpy as jnp
from jax import lax
from jax.experimental import pallas as pl
from jax.experimental.pallas import tpu as pltpu
from jax.experimental.pallas import tpu_sc as plsc

BATCH = 16384
EMBED = 32
NUM_CORES = 2
NUM_SUBCORES = 16
NUM_WORKERS = NUM_CORES * NUM_SUBCORES
B_PER_W = BATCH // NUM_WORKERS
CHUNK = 128
NUM_CHUNKS = B_PER_W // CHUNK
LANES = 16
PROJ_BLOCK = 65536


def _proj_body(t_ref, w_ref, b_ref, o_ref):
    o_ref[...] = jnp.sum(t_ref[...] * w_ref[...], axis=0) + b_ref[0]


def _project(table_t, w_col, bias):
    rows = table_t.shape[1]
    grid = (rows + PROJ_BLOCK - 1) // PROJ_BLOCK
    return pl.pallas_call(
        _proj_body,
        grid=(grid,),
        in_specs=[
            pl.BlockSpec((EMBED, PROJ_BLOCK), lambda i: (0, i)),
            pl.BlockSpec((EMBED, 1), lambda i: (0, 0)),
            pl.BlockSpec(memory_space=pltpu.SMEM),
        ],
        out_specs=pl.BlockSpec((PROJ_BLOCK,), lambda i: (i,)),
        out_shape=jax.ShapeDtypeStruct((rows,), jnp.float32),
        compiler_params=pltpu.CompilerParams(
            dimension_semantics=("parallel",)),
    )(table_t, w_col, bias)


def _make_sum_gather():
    mesh = plsc.VectorSubcoreMesh(core_axis_name="c", subcore_axis_name="s")

    def sum_gather(users, movies, proj_u, proj_m):
        @pl.kernel(
            out_type=jax.ShapeDtypeStruct((BATCH,), jnp.float32),
            mesh=mesh,
            scratch_types=[
                pltpu.VMEM((B_PER_W,), jnp.int32),
                pltpu.VMEM((B_PER_W,), jnp.int32),
                pltpu.VMEM((B_PER_W,), jnp.float32),
                pltpu.VMEM((B_PER_W,), jnp.float32),
                pltpu.VMEM((B_PER_W,), jnp.float32),
                pltpu.SemaphoreType.DMA,
            ],
        )
        def k(u_idx_hbm, m_idx_hbm, pu_hbm, pm_hbm, o_hbm,
              ui_v, mi_v, uv_v, mv_v, ov_v, sem):
            wid = lax.axis_index("s") * NUM_CORES + lax.axis_index("c")
            base = wid * B_PER_W
            pltpu.sync_copy(u_idx_hbm.at[pl.ds(base, B_PER_W)], ui_v)
            pltpu.sync_copy(m_idx_hbm.at[pl.ds(base, B_PER_W)], mi_v)
            copies = []
            for c in range(NUM_CHUNKS):
                sl = pl.ds(c * CHUNK, CHUNK)
                copies.append(pltpu.async_copy(
                    pu_hbm.at[ui_v.at[sl]], uv_v.at[sl], sem))
                copies.append(pltpu.async_copy(
                    pm_hbm.at[mi_v.at[sl]], mv_v.at[sl], sem))
            for cp in copies:
                cp.wait()

            @pl.loop(0, B_PER_W, step=LANES)
            def _(i):
                sl = pl.ds(i, LANES)
                ov_v[sl] = uv_v[sl] + mv_v[sl]

            pltpu.sync_copy(ov_v, o_hbm.at[pl.ds(base, B_PER_W)])

        return k(users, movies, proj_u, proj_m)

    return sum_gather


_sum_gather = _make_sum_gather()


def kernel(users, movies, user_table, movie_table, fc1_w, fc1_b):
    users = users.astype(jnp.int32)
    movies = movies.astype(jnp.int32)
    wu = fc1_w[0, :EMBED].reshape(EMBED, 1)
    wm = fc1_w[0, EMBED:].reshape(EMBED, 1)
    zero = jnp.zeros((1,), jnp.float32)
    proj_u = _project(user_table.T, wu, zero)
    proj_m = _project(movie_table.T, wm, fc1_b)
    out = _sum_gather(users, movies, proj_u, proj_m)
    return out.reshape(BATCH, 1)


# speedup vs baseline: 7.8523x; 1.0055x over previous
import jax
import jax.numpy as jnp
from jax import lax
from jax.experimental import pallas as pl
from jax.experimental.pallas import tpu as pltpu
from jax.experimental.pallas import tpu_sc as plsc

BATCH = 16384
EMBED = 32
NUM_CORES = 2
NUM_SUBCORES = 16
NUM_WORKERS = NUM_CORES * NUM_SUBCORES
B_PER_W = BATCH // NUM_WORKERS
CHUNK = 128
NUM_CHUNKS = B_PER_W // CHUNK
LANES = 16
PROJ_BLOCK = 131072


def _proj_body(t_ref, w_ref, b_ref, o_ref):
    o_ref[...] = jnp.sum(t_ref[...] * w_ref[...], axis=0) + b_ref[0]


def _project(table_t, w_col, bias):
    rows = table_t.shape[1]
    grid = (rows + PROJ_BLOCK - 1) // PROJ_BLOCK
    return pl.pallas_call(
        _proj_body,
        grid=(grid,),
        in_specs=[
            pl.BlockSpec((EMBED, PROJ_BLOCK), lambda i: (0, i)),
            pl.BlockSpec((EMBED, 1), lambda i: (0, 0)),
            pl.BlockSpec(memory_space=pltpu.SMEM),
        ],
        out_specs=pl.BlockSpec((PROJ_BLOCK,), lambda i: (i,)),
        out_shape=jax.ShapeDtypeStruct((rows,), jnp.float32),
        compiler_params=pltpu.CompilerParams(
            dimension_semantics=("parallel",)),
    )(table_t, w_col, bias)


def _make_sum_gather():
    mesh = plsc.VectorSubcoreMesh(core_axis_name="c", subcore_axis_name="s")

    def sum_gather(users, movies, proj_u, proj_m):
        @pl.kernel(
            out_type=jax.ShapeDtypeStruct((BATCH,), jnp.float32),
            mesh=mesh,
            scratch_types=[
                pltpu.VMEM((B_PER_W,), jnp.int32),
                pltpu.VMEM((B_PER_W,), jnp.int32),
                pltpu.VMEM((B_PER_W,), jnp.float32),
                pltpu.VMEM((B_PER_W,), jnp.float32),
                pltpu.VMEM((B_PER_W,), jnp.float32),
                pltpu.SemaphoreType.DMA,
            ],
        )
        def k(u_idx_hbm, m_idx_hbm, pu_hbm, pm_hbm, o_hbm,
              ui_v, mi_v, uv_v, mv_v, ov_v, sem):
            wid = lax.axis_index("s") * NUM_CORES + lax.axis_index("c")
            base = wid * B_PER_W
            pltpu.sync_copy(u_idx_hbm.at[pl.ds(base, B_PER_W)], ui_v)
            pltpu.sync_copy(m_idx_hbm.at[pl.ds(base, B_PER_W)], mi_v)
            copies = []
            for c in range(NUM_CHUNKS):
                sl = pl.ds(c * CHUNK, CHUNK)
                copies.append(pltpu.async_copy(
                    pu_hbm.at[ui_v.at[sl]], uv_v.at[sl], sem))
                copies.append(pltpu.async_copy(
                    pm_hbm.at[mi_v.at[sl]], mv_v.at[sl], sem))
            for cp in copies:
                cp.wait()

            @pl.loop(0, B_PER_W, step=LANES)
            def _(i):
                sl = pl.ds(i, LANES)
                ov_v[sl] = uv_v[sl] + mv_v[sl]

            pltpu.sync_copy(ov_v, o_hbm.at[pl.ds(base, B_PER_W)])

        return k(users, movies, proj_u, proj_m)

    return sum_gather


_sum_gather = _make_sum_gather()


def kernel(users, movies, user_table, movie_table, fc1_w, fc1_b):
    users = users.astype(jnp.int32)
    movies = movies.astype(jnp.int32)
    wu = fc1_w[0, :EMBED].reshape(EMBED, 1)
    wm = fc1_w[0, EMBED:].reshape(EMBED, 1)
    zero = jnp.zeros((1,), jnp.float32)
    proj_u = _project(user_table.T, wu, zero)
    proj_m = _project(movie_table.T, wm, fc1_b)
    out = _sum_gather(users, movies, proj_u, proj_m)
    return out.reshape(BATCH, 1)


# speedup vs baseline: 8.3160x; 1.0591x over previous
import jax
import jax.numpy as jnp
from jax import lax
from jax.experimental import pallas as pl
from jax.experimental.pallas import tpu as pltpu
from jax.experimental.pallas import tpu_sc as plsc

BATCH = 16384
EMBED = 32
NUM_CORES = 2
NUM_SUBCORES = 16
NUM_WORKERS = NUM_CORES * NUM_SUBCORES
B_PER_W = BATCH // NUM_WORKERS
CHUNK = 128
NUM_CHUNKS = B_PER_W // CHUNK
LANES = 16
PROJ_BLOCK = 131072


def _proj_body(tu_ref, tm_ref, wu_ref, wm_ref, b_ref, ou_ref, om_ref):
    ou_ref[...] = jnp.sum(tu_ref[...] * wu_ref[...], axis=0)
    om_ref[...] = jnp.sum(tm_ref[...] * wm_ref[...], axis=0) + b_ref[0]


def _project(ut_t, mt_t, wu, wm, bias):
    u_rows = ut_t.shape[1]
    m_rows = mt_t.shape[1]
    grid = (u_rows + PROJ_BLOCK - 1) // PROJ_BLOCK
    mb = ((m_rows + grid - 1) // grid + 1023) // 1024 * 1024
    return pl.pallas_call(
        _proj_body,
        grid=(grid,),
        in_specs=[
            pl.BlockSpec((EMBED, PROJ_BLOCK), lambda i: (0, i)),
            pl.BlockSpec((EMBED, mb), lambda i: (0, i)),
            pl.BlockSpec((EMBED, 1), lambda i: (0, 0)),
            pl.BlockSpec((EMBED, 1), lambda i: (0, 0)),
            pl.BlockSpec(memory_space=pltpu.SMEM),
        ],
        out_specs=[
            pl.BlockSpec((PROJ_BLOCK,), lambda i: (i,)),
            pl.BlockSpec((mb,), lambda i: (i,)),
        ],
        out_shape=[
            jax.ShapeDtypeStruct((u_rows,), jnp.float32),
            jax.ShapeDtypeStruct((m_rows,), jnp.float32),
        ],
        compiler_params=pltpu.CompilerParams(
            dimension_semantics=("parallel",)),
    )(ut_t, mt_t, wu, wm, bias)


def _make_sum_gather():
    mesh = plsc.VectorSubcoreMesh(core_axis_name="c", subcore_axis_name="s")

    def sum_gather(users, movies, proj_u, proj_m):
        @pl.kernel(
            out_type=jax.ShapeDtypeStruct((BATCH,), jnp.float32),
            mesh=mesh,
            scratch_types=[
                pltpu.VMEM((B_PER_W,), jnp.int32),
                pltpu.VMEM((B_PER_W,), jnp.int32),
                pltpu.VMEM((B_PER_W,), jnp.float32),
                pltpu.VMEM((B_PER_W,), jnp.float32),
                pltpu.VMEM((B_PER_W,), jnp.float32),
                pltpu.SemaphoreType.DMA,
            ],
        )
        def k(u_idx_hbm, m_idx_hbm, pu_hbm, pm_hbm, o_hbm,
              ui_v, mi_v, uv_v, mv_v, ov_v, sem):
            wid = lax.axis_index("s") * NUM_CORES + lax.axis_index("c")
            base = wid * B_PER_W
            pltpu.sync_copy(u_idx_hbm.at[pl.ds(base, B_PER_W)], ui_v)
            pltpu.sync_copy(m_idx_hbm.at[pl.ds(base, B_PER_W)], mi_v)
            copies = []
            for c in range(NUM_CHUNKS):
                sl = pl.ds(c * CHUNK, CHUNK)
                copies.append(pltpu.async_copy(
                    pu_hbm.at[ui_v.at[sl]], uv_v.at[sl], sem))
                copies.append(pltpu.async_copy(
                    pm_hbm.at[mi_v.at[sl]], mv_v.at[sl], sem))
            for cp in copies:
                cp.wait()

            @pl.loop(0, B_PER_W, step=LANES)
            def _(i):
                sl = pl.ds(i, LANES)
                ov_v[sl] = uv_v[sl] + mv_v[sl]

            pltpu.sync_copy(ov_v, o_hbm.at[pl.ds(base, B_PER_W)])

        return k(users, movies, proj_u, proj_m)

    return sum_gather


_sum_gather = _make_sum_gather()


def kernel(users, movies, user_table, movie_table, fc1_w, fc1_b):
    users = users.astype(jnp.int32)
    movies = movies.astype(jnp.int32)
    wu = fc1_w[0, :EMBED].reshape(EMBED, 1)
    wm = fc1_w[0, EMBED:].reshape(EMBED, 1)
    proj_u, proj_m = _project(user_table.T, movie_table.T, wu, wm, fc1_b)
    out = _sum_gather(users, movies, proj_u, proj_m)
    return out.reshape(BATCH, 1)
